# baseline (speedup 1.0000x reference)
import functools
import math

import jax
import jax.numpy as jnp
from jax import lax
from jax.experimental import pallas as pl
from jax.experimental.pallas import tpu as pltpu
from jax.experimental.pallas import tpu_sc as plsc

_N = 50000
_E = 800000
_D = 64
_H = 32
_ALPHA = 0.1
_THETA = 0.5
_LAYERS = 4

_NC = 2
_NS = 16
_CHUNK = 128
_INNER = 2
_BPT = 196
_CPT = _BPT * _INNER
_EPT = _CPT * _CHUNK
_EPAD = _EPT * _NS
_EB = _EPAD // _CHUNK
_NPT = 3136
_NPAD = _NPT * _NS
_ZR = 196
_BN = 2000


def _segment_sum_sc(x2, src2, dst2):
    mesh = plsc.VectorSubcoreMesh(core_axis_name="c", subcore_axis_name="s")

    @functools.partial(
        pl.kernel,
        out_type=jax.ShapeDtypeStruct((_NC, _N, _H), jnp.float32),
        mesh=mesh,
        scratch_types=[
            pltpu.VMEM_SHARED((_NPAD, _H), jnp.float32),
            pltpu.VMEM((_ZR, _H), jnp.float32),
            pltpu.VMEM((_INNER * _CHUNK,), jnp.int32),
            pltpu.VMEM((_INNER * _CHUNK,), jnp.int32),
            pltpu.VMEM((_INNER * _CHUNK, _H), jnp.float32),
            pltpu.SemaphoreType.DMA,
        ],
        compiler_params=pltpu.CompilerParams(use_tc_tiling_on_sc=False),
    )
    def run(x2_hbm, src_hbm, dst_hbm, out_hbm, acc, zbuf, src_v, dst_v, rows_v, sem):
        c = lax.axis_index("c")
        s = lax.axis_index("s")
        zero16 = jnp.zeros((16,), jnp.float32)

        @pl.loop(0, _ZR)
        def _(i):
            zbuf[i, pl.ds(0, 16)] = zero16
            zbuf[i, pl.ds(16, 16)] = zero16

        @pl.loop(0, _NPT // _ZR)
        def _(j):
            pltpu.sync_copy(zbuf, acc.at[pl.ds(s * _NPT + j * _ZR, _ZR), :])

        plsc.subcore_barrier()

        blk = _INNER * _CHUNK

        @pl.loop(0, _BPT)
        def _(b):
            e0 = s * _EPT + b * blk
            pltpu.sync_copy(src_hbm.at[c, pl.ds(e0, blk)], src_v)
            pltpu.sync_copy(dst_hbm.at[pl.ds(e0, blk)], dst_v)
            pltpu.async_copy(x2_hbm.at[src_v], rows_v, sem).wait()
            pltpu.sync_copy(rows_v, acc.at[dst_v], add=True)

        plsc.subcore_barrier()

        last = _N - (_NS - 1) * _NPT

        @pl.when(s < _NS - 1)
        def _():
            pltpu.sync_copy(acc.at[pl.ds(s * _NPT, _NPT), :],
                            out_hbm.at[c, pl.ds(s * _NPT, _NPT), :])

        @pl.when(s == _NS - 1)
        def _():
            pltpu.sync_copy(acc.at[pl.ds((_NS - 1) * _NPT, last), :],
                            out_hbm.at[c, pl.ds((_NS - 1) * _NPT, last), :])

    return run(x2, src2, dst2)


def _proj_tc(input_x, W0, b0):
    def body(x_ref, w_ref, b_ref, out_ref):
        y = jnp.dot(x_ref[...], w_ref[...], preferred_element_type=jnp.float32)
        y = jnp.maximum(y + b_ref[...], 0.0)
        out_ref[0] = y[:, :_H]
        out_ref[1] = y[:, _H:]

    return pl.pallas_call(
        body,
        grid=(_N // _BN,),
        in_specs=[
            pl.BlockSpec((_BN, _D), lambda i: (i, 0)),
            pl.BlockSpec((_D, _D), lambda i: (0, 0)),
            pl.BlockSpec((1, _D), lambda i: (0, 0)),
        ],
        out_specs=pl.BlockSpec((_NC, _BN, _H), lambda i: (0, i, 0)),
        out_shape=jax.ShapeDtypeStruct((_NC, _N, _H), jnp.float32),
    )(input_x, W0, b0.reshape(1, _D))


def _combine_tc(agg2, x02, w, beta):
    def body(a_ref, x0_ref, w_ref, out_ref):
        agg = jnp.concatenate([a_ref[0], a_ref[1]], axis=1)
        x0 = jnp.concatenate([x0_ref[0], x0_ref[1]], axis=1)
        h = (1.0 - _ALPHA) * agg + _ALPHA * x0
        hw = jnp.dot(h, w_ref[...], preferred_element_type=jnp.float32)
        y = jnp.maximum((1.0 - beta) * h + beta * hw, 0.0)
        out_ref[0] = y[:, :_H]
        out_ref[1] = y[:, _H:]

    return pl.pallas_call(
        body,
        grid=(_N // _BN,),
        in_specs=[
            pl.BlockSpec((_NC, _BN, _H), lambda i: (0, i, 0)),
            pl.BlockSpec((_NC, _BN, _H), lambda i: (0, i, 0)),
            pl.BlockSpec((_D, _D), lambda i: (0, 0)),
        ],
        out_specs=pl.BlockSpec((_NC, _BN, _H), lambda i: (0, i, 0)),
        out_shape=jax.ShapeDtypeStruct((_NC, _N, _H), jnp.float32),
    )(agg2, x02, w)


def _final_tc(agg2, x02, w, W1, b1, beta):
    grid = _N // _BN

    def body(a_ref, x0_ref, w_ref, w1_ref, b1_ref, out_ref, acc_ref):
        i = pl.program_id(0)
        agg = jnp.concatenate([a_ref[0], a_ref[1]], axis=1)
        x0 = jnp.concatenate([x0_ref[0], x0_ref[1]], axis=1)
        h = (1.0 - _ALPHA) * agg + _ALPHA * x0
        hw = jnp.dot(h, w_ref[...], preferred_element_type=jnp.float32)
        y = jnp.maximum((1.0 - beta) * h + beta * hw, 0.0)
        part = jnp.sum(y, axis=0, keepdims=True)

        @pl.when(i == 0)
        def _():
            acc_ref[...] = part

        @pl.when(i > 0)
        def _():
            acc_ref[...] = acc_ref[...] + part

        @pl.when(i == grid - 1)
        def _():
            out_ref[...] = (
                jnp.dot(acc_ref[...] * (1.0 / _N), w1_ref[...],
                        preferred_element_type=jnp.float32) + b1_ref[...]
            )

    return pl.pallas_call(
        body,
        grid=(grid,),
        in_specs=[
            pl.BlockSpec((_NC, _BN, _H), lambda i: (0, i, 0)),
            pl.BlockSpec((_NC, _BN, _H), lambda i: (0, i, 0)),
            pl.BlockSpec((_D, _D), lambda i: (0, 0)),
            pl.BlockSpec((_D, _D), lambda i: (0, 0)),
            pl.BlockSpec((1, _D), lambda i: (0, 0)),
        ],
        out_specs=pl.BlockSpec((1, _D), lambda i: (0, 0)),
        out_shape=jax.ShapeDtypeStruct((1, _D), jnp.float32),
        scratch_shapes=[pltpu.VMEM((1, _D), jnp.float32)],
    )(agg2, x02, w, W1, b1.reshape(1, _D))


def kernel(input_x, edge_index, W0, b0, W1, b1, conv_w):
    src = edge_index[0]
    dst = edge_index[1]
    pad = _EPAD - _E
    srcp = jnp.concatenate([src, jnp.zeros((pad,), jnp.int32)])
    dstp = jnp.concatenate([dst, jnp.full((pad,), _N, jnp.int32)])
    src2 = jnp.stack([srcp, srcp + _N])
    dst2 = dstp

    x2 = _proj_tc(input_x, W0, b0)
    x02 = x2
    y = None
    for layer in range(_LAYERS):
        beta = math.log(_THETA / (layer + 1) + 1.0)
        agg2 = _segment_sum_sc(x2.reshape(_NC * _N, _H), src2, dst2)
        if layer < _LAYERS - 1:
            x2 = _combine_tc(agg2, x02, conv_w[layer], beta)
        else:
            y = _final_tc(agg2, x02, conv_w[layer], W1, b1, beta)
    return y.reshape(_D)


# speedup vs baseline: 1.1044x; 1.1044x over previous
import functools
import math

import jax
import jax.numpy as jnp
from jax import lax
from jax.experimental import pallas as pl
from jax.experimental.pallas import tpu as pltpu
from jax.experimental.pallas import tpu_sc as plsc

_N = 50000
_E = 800000
_D = 64
_H = 32
_ALPHA = 0.1
_THETA = 0.5
_LAYERS = 4

_NC = 2
_NS = 16
_CHUNK = 128
_INNER = 2
_BPT = 200
_CPT = _BPT * _INNER
_EPT = _CPT * _CHUNK
_EPAD = _EPT * _NS
_EB = _EPAD // _CHUNK
_NPT = 3136
_NPAD = _NPT * _NS
_ZR = 112
_BN = 2000


def _segment_sum_sc(x2, src2, dst2):
    mesh = plsc.VectorSubcoreMesh(core_axis_name="c", subcore_axis_name="s")

    @functools.partial(
        pl.kernel,
        out_type=jax.ShapeDtypeStruct((_NC, _N, _H), jnp.float32),
        mesh=mesh,
        scratch_types=[
            pltpu.VMEM_SHARED((_NPAD, _H), jnp.float32),
            pltpu.VMEM((_ZR, _H), jnp.float32),
            [pltpu.VMEM((_INNER, _CHUNK), jnp.int32) for _ in range(2)],
            [pltpu.VMEM((_INNER, _CHUNK), jnp.int32) for _ in range(2)],
            [pltpu.VMEM((_INNER, _CHUNK, _H), jnp.float32) for _ in range(2)],
            pltpu.SemaphoreType.DMA,
            pltpu.SemaphoreType.DMA,
            [pltpu.SemaphoreType.DMA for _ in range(2)],
        ],
        compiler_params=pltpu.CompilerParams(use_tc_tiling_on_sc=False),
    )
    def run(x2_hbm, src_hbm, dst_hbm, out_hbm, acc, zbuf, isv, idv, rv,
            sem_i, sem_g, sem_s):
        c = lax.axis_index("c")
        s = lax.axis_index("s")
        zero16 = jnp.zeros((16,), jnp.float32)

        @pl.loop(0, _ZR)
        def _(i):
            zbuf[i, pl.ds(0, 16)] = zero16
            zbuf[i, pl.ds(16, 16)] = zero16

        @pl.loop(0, _NPT // _ZR)
        def _(j):
            pltpu.sync_copy(zbuf, acc.at[pl.ds(s * _NPT + j * _ZR, _ZR), :])

        plsc.subcore_barrier()

        base = s * _CPT

        def fire_idx(b, p):
            row0 = base + b * _INNER
            pltpu.async_copy(src_hbm.at[c, pl.ds(row0, _INNER), :], isv[p], sem_i)
            pltpu.async_copy(dst_hbm.at[pl.ds(row0, _INNER), :], idv[p], sem_i)

        def wait_idx(b, p):
            row0 = base + b * _INNER
            pltpu.make_async_copy(
                src_hbm.at[c, pl.ds(row0, _INNER), :], isv[p], sem_i).wait()
            pltpu.make_async_copy(
                dst_hbm.at[pl.ds(row0, _INNER), :], idv[p], sem_i).wait()

        def drain_scatters(p):
            for j in range(_INNER):
                pltpu.make_async_copy(
                    rv[p].at[j], acc.at[idv[p].at[j]], sem_s[p]).wait()

        fire_idx(0, 0)

        @pl.loop(0, _BPT // 2)
        def _(bb):
            for p in range(2):
                b = bb * 2 + p
                wait_idx(b, p)

                for j in range(_INNER):
                    pltpu.async_copy(x2_hbm.at[isv[p].at[j]], rv[p].at[j], sem_g)

                @pl.when(b >= 1)
                def _():
                    drain_scatters(1 - p)

                @pl.when(b + 1 < _BPT)
                def _():
                    fire_idx(b + 1, 1 - p)

                for j in range(_INNER):
                    pltpu.make_async_copy(
                        x2_hbm.at[isv[p].at[j]], rv[p].at[j], sem_g).wait()
                for j in range(_INNER):
                    pltpu.async_copy(
                        rv[p].at[j], acc.at[idv[p].at[j]], sem_s[p], add=True)

        drain_scatters(1)

        plsc.subcore_barrier()

        last = _N - (_NS - 1) * _NPT

        @pl.when(s < _NS - 1)
        def _():
            pltpu.sync_copy(acc.at[pl.ds(s * _NPT, _NPT), :],
                            out_hbm.at[c, pl.ds(s * _NPT, _NPT), :])

        @pl.when(s == _NS - 1)
        def _():
            pltpu.sync_copy(acc.at[pl.ds((_NS - 1) * _NPT, last), :],
                            out_hbm.at[c, pl.ds((_NS - 1) * _NPT, last), :])

    return run(x2, src2, dst2)


def _proj_tc(input_x, W0, b0):
    def body(x_ref, w_ref, b_ref, out_ref):
        y = jnp.dot(x_ref[...], w_ref[...], preferred_element_type=jnp.float32)
        y = jnp.maximum(y + b_ref[...], 0.0)
        out_ref[0] = y[:, :_H]
        out_ref[1] = y[:, _H:]

    return pl.pallas_call(
        body,
        grid=(_N // _BN,),
        in_specs=[
            pl.BlockSpec((_BN, _D), lambda i: (i, 0)),
            pl.BlockSpec((_D, _D), lambda i: (0, 0)),
            pl.BlockSpec((1, _D), lambda i: (0, 0)),
        ],
        out_specs=pl.BlockSpec((_NC, _BN, _H), lambda i: (0, i, 0)),
        out_shape=jax.ShapeDtypeStruct((_NC, _N, _H), jnp.float32),
    )(input_x, W0, b0.reshape(1, _D))


def _combine_tc(agg2, x02, w, beta):
    def body(a_ref, x0_ref, w_ref, out_ref):
        agg = jnp.concatenate([a_ref[0], a_ref[1]], axis=1)
        x0 = jnp.concatenate([x0_ref[0], x0_ref[1]], axis=1)
        h = (1.0 - _ALPHA) * agg + _ALPHA * x0
        hw = jnp.dot(h, w_ref[...], preferred_element_type=jnp.float32)
        y = jnp.maximum((1.0 - beta) * h + beta * hw, 0.0)
        out_ref[0] = y[:, :_H]
        out_ref[1] = y[:, _H:]

    return pl.pallas_call(
        body,
        grid=(_N // _BN,),
        in_specs=[
            pl.BlockSpec((_NC, _BN, _H), lambda i: (0, i, 0)),
            pl.BlockSpec((_NC, _BN, _H), lambda i: (0, i, 0)),
            pl.BlockSpec((_D, _D), lambda i: (0, 0)),
        ],
        out_specs=pl.BlockSpec((_NC, _BN, _H), lambda i: (0, i, 0)),
        out_shape=jax.ShapeDtypeStruct((_NC, _N, _H), jnp.float32),
    )(agg2, x02, w)


def _final_tc(agg2, x02, w, W1, b1, beta):
    grid = _N // _BN

    def body(a_ref, x0_ref, w_ref, w1_ref, b1_ref, out_ref, acc_ref):
        i = pl.program_id(0)
        agg = jnp.concatenate([a_ref[0], a_ref[1]], axis=1)
        x0 = jnp.concatenate([x0_ref[0], x0_ref[1]], axis=1)
        h = (1.0 - _ALPHA) * agg + _ALPHA * x0
        hw = jnp.dot(h, w_ref[...], preferred_element_type=jnp.float32)
        y = jnp.maximum((1.0 - beta) * h + beta * hw, 0.0)
        part = jnp.sum(y, axis=0, keepdims=True)

        @pl.when(i == 0)
        def _():
            acc_ref[...] = part

        @pl.when(i > 0)
        def _():
            acc_ref[...] = acc_ref[...] + part

        @pl.when(i == grid - 1)
        def _():
            out_ref[...] = (
                jnp.dot(acc_ref[...] * (1.0 / _N), w1_ref[...],
                        preferred_element_type=jnp.float32) + b1_ref[...]
            )

    return pl.pallas_call(
        body,
        grid=(grid,),
        in_specs=[
            pl.BlockSpec((_NC, _BN, _H), lambda i: (0, i, 0)),
            pl.BlockSpec((_NC, _BN, _H), lambda i: (0, i, 0)),
            pl.BlockSpec((_D, _D), lambda i: (0, 0)),
            pl.BlockSpec((_D, _D), lambda i: (0, 0)),
            pl.BlockSpec((1, _D), lambda i: (0, 0)),
        ],
        out_specs=pl.BlockSpec((1, _D), lambda i: (0, 0)),
        out_shape=jax.ShapeDtypeStruct((1, _D), jnp.float32),
        scratch_shapes=[pltpu.VMEM((1, _D), jnp.float32)],
    )(agg2, x02, w, W1, b1.reshape(1, _D))


def kernel(input_x, edge_index, W0, b0, W1, b1, conv_w):
    src = edge_index[0]
    dst = edge_index[1]
    pad = _EPAD - _E
    srcp = jnp.concatenate([src, jnp.zeros((pad,), jnp.int32)])
    dstp = jnp.concatenate([dst, jnp.full((pad,), _N, jnp.int32)])
    src2 = jnp.stack([srcp, srcp + _N]).reshape(_NC, _EB, _CHUNK)
    dst2 = dstp.reshape(_EB, _CHUNK)

    x2 = _proj_tc(input_x, W0, b0)
    x02 = x2
    y = None
    for layer in range(_LAYERS):
        beta = math.log(_THETA / (layer + 1) + 1.0)
        agg2 = _segment_sum_sc(x2.reshape(_NC * _N, _H), src2, dst2)
        if layer < _LAYERS - 1:
            x2 = _combine_tc(agg2, x02, conv_w[layer], beta)
        else:
            y = _final_tc(agg2, x02, conv_w[layer], W1, b1, beta)
    return y.reshape(_D)


# speedup vs baseline: 2.1777x; 1.9718x over previous
import functools
import math

import jax
import jax.numpy as jnp
import numpy as np
from jax import lax
from jax.experimental import pallas as pl
from jax.experimental.pallas import tpu as pltpu
from jax.experimental.pallas import tpu_sc as plsc

_N = 50000
_E = 800000
_D = 64
_HS = 16
_ALPHA = 0.1
_THETA = 0.5
_LAYERS = 4

_NC = 2
_NS = 16
_CHUNK = 128
_INNER = 4
_BPT = 100
_CPT = _BPT * _INNER
_EPT = _CPT * _CHUNK
_EPAD = _EPT * _NS
_EB = _EPAD // _CHUNK
_NPT = 3136
_NPAD = _NPT * _NS
_ZR = 112
_TPT = _N // _NS

_NR = _N // 8
_BR = _NR
_GRID = 1


def _segment_sum_sc(x4, src2, dst2):
    mesh = plsc.VectorSubcoreMesh(core_axis_name="c", subcore_axis_name="s")

    @functools.partial(
        pl.kernel,
        out_type=jax.ShapeDtypeStruct((2 * _NC, _N, _HS), jnp.float32),
        mesh=mesh,
        scratch_types=[
            pltpu.VMEM_SHARED((_NPAD, _HS), jnp.float32),
            pltpu.VMEM_SHARED((_N, _HS), jnp.float32),
            pltpu.VMEM((_ZR, _HS), jnp.float32),
            [pltpu.VMEM((_INNER, _CHUNK), jnp.int32) for _ in range(2)],
            [pltpu.VMEM((_INNER, _CHUNK), jnp.int32) for _ in range(2)],
            [pltpu.VMEM((_INNER, _CHUNK, _HS), jnp.float32) for _ in range(2)],
            pltpu.SemaphoreType.DMA,
            pltpu.SemaphoreType.DMA,
            pltpu.SemaphoreType.DMA,
            [pltpu.SemaphoreType.DMA for _ in range(2)],
        ],
        compiler_params=pltpu.CompilerParams(use_tc_tiling_on_sc=False),
    )
    def run(x4_hbm, src_hbm, dst_hbm, out_hbm, acc, tbl, zbuf, isv, idv, rv,
            sem_t, sem_i, sem_g, sem_s):
        c = lax.axis_index("c")
        s = lax.axis_index("s")
        zero16 = jnp.zeros((16,), jnp.float32)

        @pl.loop(0, _ZR)
        def _(i):
            zbuf[i, :] = zero16

        base = s * _CPT

        def fire_idx(b, p):
            row0 = base + b * _INNER
            pltpu.async_copy(src_hbm.at[pl.ds(row0, _INNER), :], isv[p], sem_i)
            pltpu.async_copy(dst_hbm.at[pl.ds(row0, _INNER), :], idv[p], sem_i)

        def wait_idx(b, p):
            row0 = base + b * _INNER
            pltpu.make_async_copy(
                src_hbm.at[pl.ds(row0, _INNER), :], isv[p], sem_i).wait()
            pltpu.make_async_copy(
                dst_hbm.at[pl.ds(row0, _INNER), :], idv[p], sem_i).wait()

        def drain_scatters(p):
            for j in range(_INNER):
                pltpu.make_async_copy(
                    rv[p].at[j], acc.at[idv[p].at[j]], sem_s[p]).wait()

        for ps in range(2):
            q = 2 * c + ps
            pltpu.async_copy(
                x4_hbm.at[q, pl.ds(s * _TPT, _TPT), :],
                tbl.at[pl.ds(s * _TPT, _TPT), :], sem_t)
            @pl.loop(0, _NPT // _ZR)
            def _(j):
                pltpu.sync_copy(zbuf, acc.at[pl.ds(s * _NPT + j * _ZR, _ZR), :])

            pltpu.make_async_copy(
                x4_hbm.at[q, pl.ds(s * _TPT, _TPT), :],
                tbl.at[pl.ds(s * _TPT, _TPT), :], sem_t).wait()
            plsc.subcore_barrier()

            fire_idx(0, 0)

            @pl.loop(0, _BPT // 2)
            def _(bb):
                for p in range(2):
                    b = bb * 2 + p
                    wait_idx(b, p)

                    for j in range(_INNER):
                        pltpu.async_copy(tbl.at[isv[p].at[j]], rv[p].at[j], sem_g)

                    @pl.when(b >= 1)
                    def _():
                        drain_scatters(1 - p)

                    @pl.when(b + 1 < _BPT)
                    def _():
                        fire_idx(b + 1, 1 - p)

                    for j in range(_INNER):
                        pltpu.make_async_copy(
                            tbl.at[isv[p].at[j]], rv[p].at[j], sem_g).wait()
                    for j in range(_INNER):
                        pltpu.async_copy(
                            rv[p].at[j], acc.at[idv[p].at[j]], sem_s[p], add=True)

            drain_scatters(1)

            plsc.subcore_barrier()

            last = _N - (_NS - 1) * _NPT

            @pl.when(s < _NS - 1)
            def _():
                pltpu.sync_copy(acc.at[pl.ds(s * _NPT, _NPT), :],
                                out_hbm.at[q, pl.ds(s * _NPT, _NPT), :])

            @pl.when(s == _NS - 1)
            def _():
                pltpu.sync_copy(acc.at[pl.ds((_NS - 1) * _NPT, last), :],
                                out_hbm.at[q, pl.ds((_NS - 1) * _NPT, last), :])

    return run(x4, src2, dst2)


def _proj_tc(x8, w0k, b0k):
    def body(x_ref, w_ref, b_ref, out_ref):
        x = x_ref[...]
        for q in range(4):
            y = jnp.dot(x, w_ref[q], preferred_element_type=jnp.float32)
            out_ref[q] = jnp.maximum(y + b_ref[q], 0.0)

    return pl.pallas_call(
        body,
        grid=(_GRID,),
        in_specs=[
            pl.BlockSpec((_BR, 8 * _D), lambda i: (i, 0)),
            pl.BlockSpec((4, 8 * _D, 128), lambda i: (0, 0, 0)),
            pl.BlockSpec((4, 1, 128), lambda i: (0, 0, 0)),
        ],
        out_specs=pl.BlockSpec((4, _BR, 128), lambda i: (0, i, 0)),
        out_shape=jax.ShapeDtypeStruct((4, _NR, 128), jnp.float32),
    )(x8, w0k, b0k)


def _combine_tc(agg4, x04, kmat, beta):
    def body(a_ref, x0_ref, k_ref, out_ref):
        hs = [(1.0 - _ALPHA) * a_ref[q] + _ALPHA * x0_ref[q] for q in range(4)]
        for q in range(4):
            hw = jnp.dot(hs[0], k_ref[0, q], preferred_element_type=jnp.float32)
            for qp in range(1, 4):
                hw += jnp.dot(hs[qp], k_ref[qp, q],
                              preferred_element_type=jnp.float32)
            out_ref[q] = jnp.maximum((1.0 - beta) * hs[q] + beta * hw, 0.0)

    return pl.pallas_call(
        body,
        grid=(_GRID,),
        in_specs=[
            pl.BlockSpec((4, _BR, 128), lambda i: (0, i, 0)),
            pl.BlockSpec((4, _BR, 128), lambda i: (0, i, 0)),
            pl.BlockSpec((4, 4, 128, 128), lambda i: (0, 0, 0, 0)),
        ],
        out_specs=pl.BlockSpec((4, _BR, 128), lambda i: (0, i, 0)),
        out_shape=jax.ShapeDtypeStruct((4, _NR, 128), jnp.float32),
    )(agg4, x04, kmat)


def _final_tc(agg4, x04, kmat, fmat, W1, b1, beta):
    def body(a_ref, x0_ref, k_ref, f_ref, w1_ref, b1_ref, out_ref, acc_ref):
        i = pl.program_id(0)
        hs = [(1.0 - _ALPHA) * a_ref[q] + _ALPHA * x0_ref[q] for q in range(4)]
        parts = []
        for q in range(4):
            hw = jnp.dot(hs[0], k_ref[0, q], preferred_element_type=jnp.float32)
            for qp in range(1, 4):
                hw += jnp.dot(hs[qp], k_ref[qp, q],
                              preferred_element_type=jnp.float32)
            y = jnp.maximum((1.0 - beta) * hs[q] + beta * hw, 0.0)
            parts.append(jnp.sum(y, axis=0))
        part = jnp.stack(parts)

        @pl.when(i == 0)
        def _():
            acc_ref[...] = part

        @pl.when(i > 0)
        def _():
            acc_ref[...] = acc_ref[...] + part

        @pl.when(i == _GRID - 1)
        def _():
            sq = acc_ref[...]
            y64 = jnp.concatenate(
                [jnp.dot(sq[q:q + 1, :], f_ref[...],
                         preferred_element_type=jnp.float32)
                 for q in range(4)], axis=1)
            out_ref[...] = (
                jnp.dot(y64 * (1.0 / _N), w1_ref[...],
                        preferred_element_type=jnp.float32) + b1_ref[...])

    return pl.pallas_call(
        body,
        grid=(_GRID,),
        in_specs=[
            pl.BlockSpec((4, _BR, 128), lambda i: (0, i, 0)),
            pl.BlockSpec((4, _BR, 128), lambda i: (0, i, 0)),
            pl.BlockSpec((4, 4, 128, 128), lambda i: (0, 0, 0, 0)),
            pl.BlockSpec((128, _HS), lambda i: (0, 0)),
            pl.BlockSpec((_D, _D), lambda i: (0, 0)),
            pl.BlockSpec((1, _D), lambda i: (0, 0)),
        ],
        out_specs=pl.BlockSpec((1, _D), lambda i: (0, 0)),
        out_shape=jax.ShapeDtypeStruct((1, _D), jnp.float32),
        scratch_shapes=[pltpu.VMEM((4, 128), jnp.float32)],
    )(agg4, x04, kmat, fmat, W1, b1.reshape(1, _D))


def kernel(input_x, edge_index, W0, b0, W1, b1, conv_w):
    src = edge_index[0]
    dst = edge_index[1]
    pad = _EPAD - _E
    srcp = jnp.concatenate([src, jnp.zeros((pad,), jnp.int32)])
    dstp = jnp.concatenate(
        [dst, _N + jnp.arange(pad, dtype=jnp.int32) % (_NPAD - _N)])
    src2 = srcp.reshape(_EB, _CHUNK)
    dst2 = dstp.reshape(_EB, _CHUNK)

    eye8 = jnp.eye(8, dtype=jnp.float32)
    w0k = jnp.stack([jnp.kron(eye8, W0[:, 16 * q:16 * q + 16])
                     for q in range(4)])
    b0k = jnp.tile(b0.reshape(4, 1, _HS), (1, 1, 8))
    kmats = [jnp.stack([jnp.stack(
        [jnp.kron(eye8, conv_w[l, 16 * a:16 * a + 16, 16 * b:16 * b + 16])
         for b in range(4)]) for a in range(4)])
        for l in range(_LAYERS)]
    fmat = jnp.asarray(np.kron(np.ones((8, 1), np.float32),
                               np.eye(_HS, dtype=np.float32)))

    x8 = input_x.reshape(_NR, 8 * _D)
    x4 = _proj_tc(x8, w0k, b0k)
    x04 = x4
    y = None
    for layer in range(_LAYERS):
        beta = math.log(_THETA / (layer + 1) + 1.0)
        agg = _segment_sum_sc(x4.reshape(4, _N, _HS), src2, dst2)
        agg4 = agg.reshape(4, _NR, 128)
        if layer < _LAYERS - 1:
            x4 = _combine_tc(agg4, x04, kmats[layer], beta)
        else:
            y = _final_tc(agg4, x04, kmats[layer], fmat, W1, b1, beta)
    return y.reshape(_D)


# speedup vs baseline: 2.2264x; 1.0224x over previous
import functools
import math

import jax
import jax.numpy as jnp
import numpy as np
from jax import lax
from jax.experimental import pallas as pl
from jax.experimental.pallas import tpu as pltpu
from jax.experimental.pallas import tpu_sc as plsc

_N = 50000
_E = 800000
_D = 64
_HS = 16
_ALPHA = 0.1
_THETA = 0.5
_LAYERS = 4

_NC = 2
_NS = 16
_CHUNK = 128
_INNER = 5
_BPT = 80
_CPT = _BPT * _INNER
_EPT = _CPT * _CHUNK
_EPAD = _EPT * _NS
_EB = _EPAD // _CHUNK
_NPT = 3136
_NPAD = _NPT * _NS
_ZR = 112
_TPT = _N // _NS

_NR = _N // 8
_BR = _NR
_GRID = 1


def _segment_sum_sc(x4, src2, dst2):
    mesh = plsc.VectorSubcoreMesh(core_axis_name="c", subcore_axis_name="s")

    @functools.partial(
        pl.kernel,
        out_type=jax.ShapeDtypeStruct((2 * _NC, _N, _HS), jnp.float32),
        mesh=mesh,
        scratch_types=[
            pltpu.VMEM_SHARED((_NPAD, _HS), jnp.float32),
            pltpu.VMEM_SHARED((_N, _HS), jnp.float32),
            pltpu.VMEM((_ZR, _HS), jnp.float32),
            [pltpu.VMEM((_INNER, _CHUNK), jnp.int32) for _ in range(2)],
            [pltpu.VMEM((_INNER, _CHUNK), jnp.int32) for _ in range(2)],
            [pltpu.VMEM((_INNER, _CHUNK, _HS), jnp.float32) for _ in range(2)],
            pltpu.SemaphoreType.DMA,
            pltpu.SemaphoreType.DMA,
            pltpu.SemaphoreType.DMA,
            [pltpu.SemaphoreType.DMA for _ in range(2)],
        ],
        compiler_params=pltpu.CompilerParams(use_tc_tiling_on_sc=False),
    )
    def run(x4_hbm, src_hbm, dst_hbm, out_hbm, acc, tbl, zbuf, isv, idv, rv,
            sem_t, sem_i, sem_g, sem_s):
        c = lax.axis_index("c")
        s = lax.axis_index("s")
        zero16 = jnp.zeros((16,), jnp.float32)

        @pl.loop(0, _ZR)
        def _(i):
            zbuf[i, :] = zero16

        base = s * _CPT

        def fire_idx(b, p):
            row0 = base + b * _INNER
            pltpu.async_copy(src_hbm.at[pl.ds(row0, _INNER), :], isv[p], sem_i)
            pltpu.async_copy(dst_hbm.at[pl.ds(row0, _INNER), :], idv[p], sem_i)

        def wait_idx(b, p):
            row0 = base + b * _INNER
            pltpu.make_async_copy(
                src_hbm.at[pl.ds(row0, _INNER), :], isv[p], sem_i).wait()
            pltpu.make_async_copy(
                dst_hbm.at[pl.ds(row0, _INNER), :], idv[p], sem_i).wait()

        def drain_scatters(p):
            for j in range(_INNER):
                pltpu.make_async_copy(
                    rv[p].at[j], acc.at[idv[p].at[j]], sem_s[p]).wait()

        for ps in range(2):
            q = 2 * c + ps
            pltpu.async_copy(
                x4_hbm.at[q, pl.ds(s * _TPT, _TPT), :],
                tbl.at[pl.ds(s * _TPT, _TPT), :], sem_t)
            @pl.loop(0, _NPT // _ZR)
            def _(j):
                pltpu.sync_copy(zbuf, acc.at[pl.ds(s * _NPT + j * _ZR, _ZR), :])

            pltpu.make_async_copy(
                x4_hbm.at[q, pl.ds(s * _TPT, _TPT), :],
                tbl.at[pl.ds(s * _TPT, _TPT), :], sem_t).wait()
            plsc.subcore_barrier()

            fire_idx(0, 0)

            @pl.loop(0, _BPT // 2)
            def _(bb):
                for p in range(2):
                    b = bb * 2 + p
                    wait_idx(b, p)

                    for j in range(_INNER):
                        pltpu.async_copy(tbl.at[isv[p].at[j]], rv[p].at[j], sem_g)

                    @pl.when(b >= 1)
                    def _():
                        drain_scatters(1 - p)

                    @pl.when(b + 1 < _BPT)
                    def _():
                        fire_idx(b + 1, 1 - p)

                    for j in range(_INNER):
                        pltpu.make_async_copy(
                            tbl.at[isv[p].at[j]], rv[p].at[j], sem_g).wait()
                    for j in range(_INNER):
                        pltpu.async_copy(
                            rv[p].at[j], acc.at[idv[p].at[j]], sem_s[p], add=True)

            drain_scatters(1)

            plsc.subcore_barrier()

            last = _N - (_NS - 1) * _NPT

            @pl.when(s < _NS - 1)
            def _():
                pltpu.sync_copy(acc.at[pl.ds(s * _NPT, _NPT), :],
                                out_hbm.at[q, pl.ds(s * _NPT, _NPT), :])

            @pl.when(s == _NS - 1)
            def _():
                pltpu.sync_copy(acc.at[pl.ds((_NS - 1) * _NPT, last), :],
                                out_hbm.at[q, pl.ds((_NS - 1) * _NPT, last), :])

    return run(x4, src2, dst2)


def _proj_tc(x8, w0k, b0k):
    def body(x_ref, w_ref, b_ref, out_ref):
        x = x_ref[...]
        for q in range(4):
            y = jnp.dot(x, w_ref[q], preferred_element_type=jnp.float32)
            out_ref[q] = jnp.maximum(y + b_ref[q], 0.0)

    return pl.pallas_call(
        body,
        grid=(_GRID,),
        in_specs=[
            pl.BlockSpec((_BR, 8 * _D), lambda i: (i, 0)),
            pl.BlockSpec((4, 8 * _D, 128), lambda i: (0, 0, 0)),
            pl.BlockSpec((4, 1, 128), lambda i: (0, 0, 0)),
        ],
        out_specs=pl.BlockSpec((4, _BR, 128), lambda i: (0, i, 0)),
        out_shape=jax.ShapeDtypeStruct((4, _NR, 128), jnp.float32),
    )(x8, w0k, b0k)


def _combine_tc(agg4, x04, kmat, beta):
    def body(a_ref, x0_ref, k_ref, out_ref):
        hs = [(1.0 - _ALPHA) * a_ref[q] + _ALPHA * x0_ref[q] for q in range(4)]
        for q in range(4):
            hw = jnp.dot(hs[0], k_ref[0, q], preferred_element_type=jnp.float32)
            for qp in range(1, 4):
                hw += jnp.dot(hs[qp], k_ref[qp, q],
                              preferred_element_type=jnp.float32)
            out_ref[q] = jnp.maximum((1.0 - beta) * hs[q] + beta * hw, 0.0)

    return pl.pallas_call(
        body,
        grid=(_GRID,),
        in_specs=[
            pl.BlockSpec((4, _BR, 128), lambda i: (0, i, 0)),
            pl.BlockSpec((4, _BR, 128), lambda i: (0, i, 0)),
            pl.BlockSpec((4, 4, 128, 128), lambda i: (0, 0, 0, 0)),
        ],
        out_specs=pl.BlockSpec((4, _BR, 128), lambda i: (0, i, 0)),
        out_shape=jax.ShapeDtypeStruct((4, _NR, 128), jnp.float32),
    )(agg4, x04, kmat)


def _final_tc(agg4, x04, kmat, fmat, W1, b1, beta):
    def body(a_ref, x0_ref, k_ref, f_ref, w1_ref, b1_ref, out_ref, acc_ref):
        i = pl.program_id(0)
        hs = [(1.0 - _ALPHA) * a_ref[q] + _ALPHA * x0_ref[q] for q in range(4)]
        parts = []
        for q in range(4):
            hw = jnp.dot(hs[0], k_ref[0, q], preferred_element_type=jnp.float32)
            for qp in range(1, 4):
                hw += jnp.dot(hs[qp], k_ref[qp, q],
                              preferred_element_type=jnp.float32)
            y = jnp.maximum((1.0 - beta) * hs[q] + beta * hw, 0.0)
            parts.append(jnp.sum(y, axis=0))
        part = jnp.stack(parts)

        @pl.when(i == 0)
        def _():
            acc_ref[...] = part

        @pl.when(i > 0)
        def _():
            acc_ref[...] = acc_ref[...] + part

        @pl.when(i == _GRID - 1)
        def _():
            sq = acc_ref[...]
            y64 = jnp.concatenate(
                [jnp.dot(sq[q:q + 1, :], f_ref[...],
                         preferred_element_type=jnp.float32)
                 for q in range(4)], axis=1)
            out_ref[...] = (
                jnp.dot(y64 * (1.0 / _N), w1_ref[...],
                        preferred_element_type=jnp.float32) + b1_ref[...])

    return pl.pallas_call(
        body,
        grid=(_GRID,),
        in_specs=[
            pl.BlockSpec((4, _BR, 128), lambda i: (0, i, 0)),
            pl.BlockSpec((4, _BR, 128), lambda i: (0, i, 0)),
            pl.BlockSpec((4, 4, 128, 128), lambda i: (0, 0, 0, 0)),
            pl.BlockSpec((128, _HS), lambda i: (0, 0)),
            pl.BlockSpec((_D, _D), lambda i: (0, 0)),
            pl.BlockSpec((1, _D), lambda i: (0, 0)),
        ],
        out_specs=pl.BlockSpec((1, _D), lambda i: (0, 0)),
        out_shape=jax.ShapeDtypeStruct((1, _D), jnp.float32),
        scratch_shapes=[pltpu.VMEM((4, 128), jnp.float32)],
    )(agg4, x04, kmat, fmat, W1, b1.reshape(1, _D))


def kernel(input_x, edge_index, W0, b0, W1, b1, conv_w):
    src = edge_index[0]
    dst = edge_index[1]
    pad = _EPAD - _E
    srcp = jnp.concatenate([src, jnp.zeros((pad,), jnp.int32)])
    dstp = jnp.concatenate(
        [dst, _N + jnp.arange(pad, dtype=jnp.int32) % (_NPAD - _N)])
    src2 = srcp.reshape(_EB, _CHUNK)
    dst2 = dstp.reshape(_EB, _CHUNK)

    eye8 = jnp.eye(8, dtype=jnp.float32)
    w0k = jnp.stack([jnp.kron(eye8, W0[:, 16 * q:16 * q + 16])
                     for q in range(4)])
    b0k = jnp.tile(b0.reshape(4, 1, _HS), (1, 1, 8))
    cw = conv_w.reshape(_LAYERS, 4, _HS, 4, _HS)
    km = jnp.einsum("ij,lakbm->labikjm", eye8, cw)
    kmats = list(km.reshape(_LAYERS, 4, 4, 128, 128))
    fmat = jnp.asarray(np.kron(np.ones((8, 1), np.float32),
                               np.eye(_HS, dtype=np.float32)))

    x8 = input_x.reshape(_NR, 8 * _D)
    x4 = _proj_tc(x8, w0k, b0k)
    x04 = x4
    y = None
    for layer in range(_LAYERS):
        beta = math.log(_THETA / (layer + 1) + 1.0)
        agg = _segment_sum_sc(x4.reshape(4, _N, _HS), src2, dst2)
        agg4 = agg.reshape(4, _NR, 128)
        if layer < _LAYERS - 1:
            x4 = _combine_tc(agg4, x04, kmats[layer], beta)
        else:
            y = _final_tc(agg4, x04, kmats[layer], fmat, W1, b1, beta)
    return y.reshape(_D)


# speedup vs baseline: 2.2318x; 1.0024x over previous
import functools
import math

import jax
import jax.numpy as jnp
import numpy as np
from jax import lax
from jax.experimental import pallas as pl
from jax.experimental.pallas import tpu as pltpu
from jax.experimental.pallas import tpu_sc as plsc

_N = 50000
_E = 800000
_D = 64
_HS = 16
_ALPHA = 0.1
_THETA = 0.5
_LAYERS = 4

_NC = 2
_NS = 16
_CHUNK = 128
_INNER = 5
_BPT = 80
_CPT = _BPT * _INNER
_EPT = _CPT * _CHUNK
_EPAD = _EPT * _NS
_EB = _EPAD // _CHUNK
_NPT = 3136
_NPAD = _NPT * _NS
_ZR = 112
_TPT = _N // _NS

_NR = _N // 8
_BR = _NR
_GRID = 1


def _segment_sum_sc(x4, src2, dst2):
    mesh = plsc.VectorSubcoreMesh(core_axis_name="c", subcore_axis_name="s")

    @functools.partial(
        pl.kernel,
        out_type=jax.ShapeDtypeStruct((2 * _NC, _N, _HS), jnp.float32),
        mesh=mesh,
        scratch_types=[
            pltpu.VMEM_SHARED((_NPAD, _HS), jnp.float32),
            pltpu.VMEM_SHARED((_N, _HS), jnp.float32),
            pltpu.VMEM((_ZR, _HS), jnp.float32),
            [pltpu.VMEM((_INNER, _CHUNK), jnp.int32) for _ in range(2)],
            [pltpu.VMEM((_INNER, _CHUNK), jnp.int32) for _ in range(2)],
            [pltpu.VMEM((_INNER, _CHUNK, _HS), jnp.float32) for _ in range(2)],
            pltpu.SemaphoreType.DMA,
            pltpu.SemaphoreType.DMA,
            pltpu.SemaphoreType.DMA,
            pltpu.SemaphoreType.DMA,
            [pltpu.SemaphoreType.DMA for _ in range(2)],
        ],
        compiler_params=pltpu.CompilerParams(use_tc_tiling_on_sc=False),
    )
    def run(x4_hbm, src_hbm, dst_hbm, out_hbm, acc, tbl, zbuf, isv, idv, rv,
            sem_t, sem_z, sem_i, sem_g, sem_s):
        c = lax.axis_index("c")
        s = lax.axis_index("s")
        zero16 = jnp.zeros((16,), jnp.float32)

        @pl.loop(0, _ZR)
        def _(i):
            zbuf[i, :] = zero16

        base = s * _CPT

        def fire_idx(b, p):
            row0 = base + b * _INNER
            pltpu.async_copy(src_hbm.at[pl.ds(row0, _INNER), :], isv[p], sem_i)
            pltpu.async_copy(dst_hbm.at[pl.ds(row0, _INNER), :], idv[p], sem_i)

        def wait_idx(b, p):
            row0 = base + b * _INNER
            pltpu.make_async_copy(
                src_hbm.at[pl.ds(row0, _INNER), :], isv[p], sem_i).wait()
            pltpu.make_async_copy(
                dst_hbm.at[pl.ds(row0, _INNER), :], idv[p], sem_i).wait()

        def drain_scatters(p):
            for j in range(_INNER):
                pltpu.make_async_copy(
                    rv[p].at[j], acc.at[idv[p].at[j]], sem_s[p]).wait()

        for ps in range(2):
            q = 2 * c + ps
            pltpu.async_copy(
                x4_hbm.at[q, pl.ds(s * _TPT, _TPT), :],
                tbl.at[pl.ds(s * _TPT, _TPT), :], sem_t)
            @pl.loop(0, _NPT // _ZR)
            def _(j):
                pltpu.async_copy(
                    zbuf, acc.at[pl.ds(s * _NPT + j * _ZR, _ZR), :], sem_z)

            @pl.loop(0, _NPT // _ZR)
            def _(j):
                pltpu.make_async_copy(
                    zbuf, acc.at[pl.ds(s * _NPT + j * _ZR, _ZR), :], sem_z).wait()

            pltpu.make_async_copy(
                x4_hbm.at[q, pl.ds(s * _TPT, _TPT), :],
                tbl.at[pl.ds(s * _TPT, _TPT), :], sem_t).wait()
            plsc.subcore_barrier()

            fire_idx(0, 0)

            @pl.loop(0, _BPT // 2)
            def _(bb):
                for p in range(2):
                    b = bb * 2 + p
                    wait_idx(b, p)

                    for j in range(_INNER):
                        pltpu.async_copy(tbl.at[isv[p].at[j]], rv[p].at[j], sem_g)

                    @pl.when(b >= 1)
                    def _():
                        drain_scatters(1 - p)

                    @pl.when(b + 1 < _BPT)
                    def _():
                        fire_idx(b + 1, 1 - p)

                    for j in range(_INNER):
                        pltpu.make_async_copy(
                            tbl.at[isv[p].at[j]], rv[p].at[j], sem_g).wait()
                    for j in range(_INNER):
                        pltpu.async_copy(
                            rv[p].at[j], acc.at[idv[p].at[j]], sem_s[p], add=True)

            drain_scatters(1)

            plsc.subcore_barrier()

            last = _N - (_NS - 1) * _NPT

            @pl.when(s < _NS - 1)
            def _():
                pltpu.sync_copy(acc.at[pl.ds(s * _NPT, _NPT), :],
                                out_hbm.at[q, pl.ds(s * _NPT, _NPT), :])

            @pl.when(s == _NS - 1)
            def _():
                pltpu.sync_copy(acc.at[pl.ds((_NS - 1) * _NPT, last), :],
                                out_hbm.at[q, pl.ds((_NS - 1) * _NPT, last), :])

    return run(x4, src2, dst2)


def _proj_tc(x8, w0k, b0k):
    def body(x_ref, w_ref, b_ref, out_ref):
        x = x_ref[...]
        for q in range(4):
            y = jnp.dot(x, w_ref[q], preferred_element_type=jnp.float32)
            out_ref[q] = jnp.maximum(y + b_ref[q], 0.0)

    return pl.pallas_call(
        body,
        grid=(_GRID,),
        in_specs=[
            pl.BlockSpec((_BR, 8 * _D), lambda i: (i, 0)),
            pl.BlockSpec((4, 8 * _D, 128), lambda i: (0, 0, 0)),
            pl.BlockSpec((4, 1, 128), lambda i: (0, 0, 0)),
        ],
        out_specs=pl.BlockSpec((4, _BR, 128), lambda i: (0, i, 0)),
        out_shape=jax.ShapeDtypeStruct((4, _NR, 128), jnp.float32),
    )(x8, w0k, b0k)


def _combine_tc(agg4, x04, kmat, beta):
    def body(a_ref, x0_ref, k_ref, out_ref):
        hs = [(1.0 - _ALPHA) * a_ref[q] + _ALPHA * x0_ref[q] for q in range(4)]
        for q in range(4):
            hw = jnp.dot(hs[0], k_ref[0, q], preferred_element_type=jnp.float32)
            for qp in range(1, 4):
                hw += jnp.dot(hs[qp], k_ref[qp, q],
                              preferred_element_type=jnp.float32)
            out_ref[q] = jnp.maximum((1.0 - beta) * hs[q] + beta * hw, 0.0)

    return pl.pallas_call(
        body,
        grid=(_GRID,),
        in_specs=[
            pl.BlockSpec((4, _BR, 128), lambda i: (0, i, 0)),
            pl.BlockSpec((4, _BR, 128), lambda i: (0, i, 0)),
            pl.BlockSpec((4, 4, 128, 128), lambda i: (0, 0, 0, 0)),
        ],
        out_specs=pl.BlockSpec((4, _BR, 128), lambda i: (0, i, 0)),
        out_shape=jax.ShapeDtypeStruct((4, _NR, 128), jnp.float32),
    )(agg4, x04, kmat)


def _final_tc(agg4, x04, kmat, fmat, W1, b1, beta):
    def body(a_ref, x0_ref, k_ref, f_ref, w1_ref, b1_ref, out_ref, acc_ref):
        i = pl.program_id(0)
        hs = [(1.0 - _ALPHA) * a_ref[q] + _ALPHA * x0_ref[q] for q in range(4)]
        parts = []
        for q in range(4):
            hw = jnp.dot(hs[0], k_ref[0, q], preferred_element_type=jnp.float32)
            for qp in range(1, 4):
                hw += jnp.dot(hs[qp], k_ref[qp, q],
                              preferred_element_type=jnp.float32)
            y = jnp.maximum((1.0 - beta) * hs[q] + beta * hw, 0.0)
            parts.append(jnp.sum(y, axis=0))
        part = jnp.stack(parts)

        @pl.when(i == 0)
        def _():
            acc_ref[...] = part

        @pl.when(i > 0)
        def _():
            acc_ref[...] = acc_ref[...] + part

        @pl.when(i == _GRID - 1)
        def _():
            sq = acc_ref[...]
            y64 = jnp.concatenate(
                [jnp.dot(sq[q:q + 1, :], f_ref[...],
                         preferred_element_type=jnp.float32)
                 for q in range(4)], axis=1)
            out_ref[...] = (
                jnp.dot(y64 * (1.0 / _N), w1_ref[...],
                        preferred_element_type=jnp.float32) + b1_ref[...])

    return pl.pallas_call(
        body,
        grid=(_GRID,),
        in_specs=[
            pl.BlockSpec((4, _BR, 128), lambda i: (0, i, 0)),
            pl.BlockSpec((4, _BR, 128), lambda i: (0, i, 0)),
            pl.BlockSpec((4, 4, 128, 128), lambda i: (0, 0, 0, 0)),
            pl.BlockSpec((128, _HS), lambda i: (0, 0)),
            pl.BlockSpec((_D, _D), lambda i: (0, 0)),
            pl.BlockSpec((1, _D), lambda i: (0, 0)),
        ],
        out_specs=pl.BlockSpec((1, _D), lambda i: (0, 0)),
        out_shape=jax.ShapeDtypeStruct((1, _D), jnp.float32),
        scratch_shapes=[pltpu.VMEM((4, 128), jnp.float32)],
    )(agg4, x04, kmat, fmat, W1, b1.reshape(1, _D))


def kernel(input_x, edge_index, W0, b0, W1, b1, conv_w):
    src = edge_index[0]
    dst = edge_index[1]
    pad = _EPAD - _E
    srcp = jnp.concatenate([src, jnp.zeros((pad,), jnp.int32)])
    dstp = jnp.concatenate(
        [dst, _N + jnp.arange(pad, dtype=jnp.int32) % (_NPAD - _N)])
    src2 = srcp.reshape(_EB, _CHUNK)
    dst2 = dstp.reshape(_EB, _CHUNK)

    eye8 = jnp.eye(8, dtype=jnp.float32)
    w0k = jnp.stack([jnp.kron(eye8, W0[:, 16 * q:16 * q + 16])
                     for q in range(4)])
    b0k = jnp.tile(b0.reshape(4, 1, _HS), (1, 1, 8))
    cw = conv_w.reshape(_LAYERS, 4, _HS, 4, _HS)
    km = jnp.einsum("ij,lakbm->labikjm", eye8, cw)
    kmats = list(km.reshape(_LAYERS, 4, 4, 128, 128))
    fmat = jnp.asarray(np.kron(np.ones((8, 1), np.float32),
                               np.eye(_HS, dtype=np.float32)))

    x8 = input_x.reshape(_NR, 8 * _D)
    x4 = _proj_tc(x8, w0k, b0k)
    x04 = x4
    y = None
    for layer in range(_LAYERS):
        beta = math.log(_THETA / (layer + 1) + 1.0)
        agg = _segment_sum_sc(x4.reshape(4, _N, _HS), src2, dst2)
        agg4 = agg.reshape(4, _NR, 128)
        if layer < _LAYERS - 1:
            x4 = _combine_tc(agg4, x04, kmats[layer], beta)
        else:
            y = _final_tc(agg4, x04, kmats[layer], fmat, W1, b1, beta)
    return y.reshape(_D)


# speedup vs baseline: 2.4157x; 1.0824x over previous
import functools
import math

import jax
import jax.numpy as jnp
import numpy as np
from jax import lax
from jax.experimental import pallas as pl
from jax.experimental.pallas import tpu as pltpu
from jax.experimental.pallas import tpu_sc as plsc

_N = 50000
_E = 800000
_D = 64
_HS = 16
_ALPHA = 0.1
_THETA = 0.5
_LAYERS = 4

_NC = 2
_NS = 16
_CHUNK = 128
_INNER = 5
_BPT = 80
_CPT = _BPT * _INNER
_EPT = _CPT * _CHUNK
_EPAD = _EPT * _NS
_EB = _EPAD // _CHUNK
_NPT = 3136
_NPAD = _NPT * _NS
_ZR = 112
_TPT = _NPAD // _NS

_NR = _N // 8
_NRP = _NPAD // 8
_BR = 784
_GRID = _NRP // _BR


def _segment_sum_sc(x4, src2, dst2):
    mesh = plsc.VectorSubcoreMesh(core_axis_name="c", subcore_axis_name="s")

    @functools.partial(
        pl.kernel,
        out_type=jax.ShapeDtypeStruct((2 * _NC, _NPAD, _HS), jnp.float32),
        mesh=mesh,
        scratch_types=[
            pltpu.VMEM_SHARED((_NPAD, _HS), jnp.float32),
            pltpu.VMEM_SHARED((_NPAD, _HS), jnp.float32),
            pltpu.VMEM((_ZR, _HS), jnp.float32),
            [pltpu.VMEM((_INNER, _CHUNK), jnp.int32) for _ in range(2)],
            [pltpu.VMEM((_INNER, _CHUNK), jnp.int32) for _ in range(2)],
            [pltpu.VMEM((_INNER, _CHUNK, _HS), jnp.float32) for _ in range(2)],
            pltpu.SemaphoreType.DMA,
            pltpu.SemaphoreType.DMA,
            pltpu.SemaphoreType.DMA,
            pltpu.SemaphoreType.DMA,
            [pltpu.SemaphoreType.DMA for _ in range(2)],
        ],
        compiler_params=pltpu.CompilerParams(use_tc_tiling_on_sc=False),
    )
    def run(x4_hbm, src_hbm, dst_hbm, out_hbm, acc, tbl, zbuf, isv, idv, rv,
            sem_t, sem_z, sem_i, sem_g, sem_s):
        c = lax.axis_index("c")
        s = lax.axis_index("s")
        zero16 = jnp.zeros((16,), jnp.float32)

        @pl.loop(0, _ZR)
        def _(i):
            zbuf[i, :] = zero16

        base = s * _CPT

        def fire_idx(b, p):
            row0 = base + b * _INNER
            pltpu.async_copy(src_hbm.at[pl.ds(row0, _INNER), :], isv[p], sem_i)
            pltpu.async_copy(dst_hbm.at[pl.ds(row0, _INNER), :], idv[p], sem_i)

        def wait_idx(b, p):
            row0 = base + b * _INNER
            pltpu.make_async_copy(
                src_hbm.at[pl.ds(row0, _INNER), :], isv[p], sem_i).wait()
            pltpu.make_async_copy(
                dst_hbm.at[pl.ds(row0, _INNER), :], idv[p], sem_i).wait()

        def drain_scatters(p):
            for j in range(_INNER):
                pltpu.make_async_copy(
                    rv[p].at[j], acc.at[idv[p].at[j]], sem_s[p]).wait()

        for ps in range(2):
            q = 2 * c + ps
            pltpu.async_copy(
                x4_hbm.at[q, pl.ds(s * _TPT, _TPT), :],
                tbl.at[pl.ds(s * _TPT, _TPT), :], sem_t)
            @pl.loop(0, _NPT // _ZR)
            def _(j):
                pltpu.async_copy(
                    zbuf, acc.at[pl.ds(s * _NPT + j * _ZR, _ZR), :], sem_z)

            @pl.loop(0, _NPT // _ZR)
            def _(j):
                pltpu.make_async_copy(
                    zbuf, acc.at[pl.ds(s * _NPT + j * _ZR, _ZR), :], sem_z).wait()

            pltpu.make_async_copy(
                x4_hbm.at[q, pl.ds(s * _TPT, _TPT), :],
                tbl.at[pl.ds(s * _TPT, _TPT), :], sem_t).wait()
            plsc.subcore_barrier()

            fire_idx(0, 0)

            @pl.loop(0, _BPT // 2)
            def _(bb):
                for p in range(2):
                    b = bb * 2 + p
                    wait_idx(b, p)

                    for j in range(_INNER):
                        pltpu.async_copy(tbl.at[isv[p].at[j]], rv[p].at[j], sem_g)

                    @pl.when(b >= 1)
                    def _():
                        drain_scatters(1 - p)

                    @pl.when(b + 1 < _BPT)
                    def _():
                        fire_idx(b + 1, 1 - p)

                    for j in range(_INNER):
                        pltpu.make_async_copy(
                            tbl.at[isv[p].at[j]], rv[p].at[j], sem_g).wait()
                    for j in range(_INNER):
                        pltpu.async_copy(
                            rv[p].at[j], acc.at[idv[p].at[j]], sem_s[p], add=True)

            drain_scatters(1)

            plsc.subcore_barrier()

            pltpu.sync_copy(acc.at[pl.ds(s * _NPT, _NPT), :],
                            out_hbm.at[q, pl.ds(s * _NPT, _NPT), :])

    return run(x4, src2, dst2)


def _proj_tc(x8, w0k, b0k):
    def body(x_ref, w_ref, b_ref, out_ref):
        x = x_ref[...]
        for q in range(4):
            y = jnp.dot(x, w_ref[q], preferred_element_type=jnp.float32)
            out_ref[q] = jnp.maximum(y + b_ref[q], 0.0)

    return pl.pallas_call(
        body,
        grid=(_GRID,),
        in_specs=[
            pl.BlockSpec((_BR, 8 * _D), lambda i: (i, 0)),
            pl.BlockSpec((4, 8 * _D, 128), lambda i: (0, 0, 0)),
            pl.BlockSpec((4, 1, 128), lambda i: (0, 0, 0)),
        ],
        out_specs=pl.BlockSpec((4, _BR, 128), lambda i: (0, i, 0)),
        out_shape=jax.ShapeDtypeStruct((4, _NRP, 128), jnp.float32),
    )(x8, w0k, b0k)


def _combine_tc(agg4, x04, kmat, beta):
    def body(a_ref, x0_ref, k_ref, out_ref):
        hs = [(1.0 - _ALPHA) * a_ref[q] + _ALPHA * x0_ref[q] for q in range(4)]
        for q in range(4):
            hw = jnp.dot(hs[0], k_ref[0, q], preferred_element_type=jnp.float32)
            for qp in range(1, 4):
                hw += jnp.dot(hs[qp], k_ref[qp, q],
                              preferred_element_type=jnp.float32)
            out_ref[q] = jnp.maximum((1.0 - beta) * hs[q] + beta * hw, 0.0)

    return pl.pallas_call(
        body,
        grid=(_GRID,),
        in_specs=[
            pl.BlockSpec((4, _BR, 128), lambda i: (0, i, 0)),
            pl.BlockSpec((4, _BR, 128), lambda i: (0, i, 0)),
            pl.BlockSpec((4, 4, 128, 128), lambda i: (0, 0, 0, 0)),
        ],
        out_specs=pl.BlockSpec((4, _BR, 128), lambda i: (0, i, 0)),
        out_shape=jax.ShapeDtypeStruct((4, _NRP, 128), jnp.float32),
    )(agg4, x04, kmat)


def _final_tc(agg4, x04, kmat, fmat, W1, b1, beta):
    def body(a_ref, x0_ref, k_ref, f_ref, w1_ref, b1_ref, out_ref, acc_ref):
        i = pl.program_id(0)
        rows = i * _BR + jax.lax.broadcasted_iota(jnp.int32, (_BR, 128), 0)
        valid = rows < _NR
        hs = [(1.0 - _ALPHA) * a_ref[q] + _ALPHA * x0_ref[q] for q in range(4)]
        parts = []
        for q in range(4):
            hw = jnp.dot(hs[0], k_ref[0, q], preferred_element_type=jnp.float32)
            for qp in range(1, 4):
                hw += jnp.dot(hs[qp], k_ref[qp, q],
                              preferred_element_type=jnp.float32)
            y = jnp.maximum((1.0 - beta) * hs[q] + beta * hw, 0.0)
            parts.append(jnp.sum(jnp.where(valid, y, 0.0), axis=0))
        part = jnp.stack(parts)

        @pl.when(i == 0)
        def _():
            acc_ref[...] = part

        @pl.when(i > 0)
        def _():
            acc_ref[...] = acc_ref[...] + part

        @pl.when(i == _GRID - 1)
        def _():
            sq = acc_ref[...]
            y64 = jnp.concatenate(
                [jnp.dot(sq[q:q + 1, :], f_ref[...],
                         preferred_element_type=jnp.float32)
                 for q in range(4)], axis=1)
            out_ref[...] = (
                jnp.dot(y64 * (1.0 / _N), w1_ref[...],
                        preferred_element_type=jnp.float32) + b1_ref[...])

    return pl.pallas_call(
        body,
        grid=(_GRID,),
        in_specs=[
            pl.BlockSpec((4, _BR, 128), lambda i: (0, i, 0)),
            pl.BlockSpec((4, _BR, 128), lambda i: (0, i, 0)),
            pl.BlockSpec((4, 4, 128, 128), lambda i: (0, 0, 0, 0)),
            pl.BlockSpec((128, _HS), lambda i: (0, 0)),
            pl.BlockSpec((_D, _D), lambda i: (0, 0)),
            pl.BlockSpec((1, _D), lambda i: (0, 0)),
        ],
        out_specs=pl.BlockSpec((1, _D), lambda i: (0, 0)),
        out_shape=jax.ShapeDtypeStruct((1, _D), jnp.float32),
        scratch_shapes=[pltpu.VMEM((4, 128), jnp.float32)],
    )(agg4, x04, kmat, fmat, W1, b1.reshape(1, _D))


def kernel(input_x, edge_index, W0, b0, W1, b1, conv_w):
    src = edge_index[0]
    dst = edge_index[1]
    pad = _EPAD - _E
    srcp = jnp.concatenate([src, jnp.zeros((pad,), jnp.int32)])
    dstp = jnp.concatenate(
        [dst, _N + jnp.arange(pad, dtype=jnp.int32) % (_NPAD - _N)])
    src2 = srcp.reshape(_EB, _CHUNK)
    dst2 = dstp.reshape(_EB, _CHUNK)

    eye8 = jnp.eye(8, dtype=jnp.float32)
    w0k = jnp.stack([jnp.kron(eye8, W0[:, 16 * q:16 * q + 16])
                     for q in range(4)])
    b0k = jnp.tile(b0.reshape(4, 1, _HS), (1, 1, 8))
    cw = conv_w.reshape(_LAYERS, 4, _HS, 4, _HS)
    km = jnp.einsum("ij,lakbm->labikjm", eye8, cw)
    kmats = list(km.reshape(_LAYERS, 4, 4, 128, 128))
    fmat = jnp.asarray(np.kron(np.ones((8, 1), np.float32),
                               np.eye(_HS, dtype=np.float32)))

    x8 = jnp.pad(input_x, ((0, _NPAD - _N), (0, 0))).reshape(_NRP, 8 * _D)
    x4 = _proj_tc(x8, w0k, b0k)
    x04 = x4
    y = None
    for layer in range(_LAYERS):
        beta = math.log(_THETA / (layer + 1) + 1.0)
        agg = _segment_sum_sc(x4.reshape(4, _NPAD, _HS), src2, dst2)
        agg4 = agg.reshape(4, _NRP, 128)
        if layer < _LAYERS - 1:
            x4 = _combine_tc(agg4, x04, kmats[layer], beta)
        else:
            y = _final_tc(agg4, x04, kmats[layer], fmat, W1, b1, beta)
    return y.reshape(_D)
